# baseline (speedup 1.0000x reference)
import functools

import jax
import jax.numpy as jnp
from jax import lax
from jax.experimental import pallas as pl
from jax.experimental.pallas import tpu as pltpu
from jax.experimental.pallas import tpu_sc as plsc

_NW = 32
_L = 16


@functools.cache
def _build(B, T, N, C, K, LG):
    G = 1 << LG
    d = (N * C) // G
    NP = B * K * G
    PPW = NP // _NW
    NG = PPW // _L
    AG = (B * K) // _L

    mesh = plsc.VectorSubcoreMesh(core_axis_name="c", subcore_axis_name="s")

    @functools.partial(
        pl.kernel,
        mesh=mesh,
        compiler_params=pltpu.CompilerParams(needs_layout_passes=False),
        out_type=[
            jax.ShapeDtypeStruct((NP, d), jnp.float32),
            jax.ShapeDtypeStruct((B * K, C), jnp.float32),
        ],
        scratch_types=[
            pltpu.VMEM((B * K,), jnp.int32),
            pltpu.VMEM((_L, d), jnp.float32),
            pltpu.VMEM((_L, d), jnp.float32),
            pltpu.VMEM((_L, C), jnp.float32),
            pltpu.SemaphoreType.DMA,
            pltpu.SemaphoreType.DMA,
            pltpu.SemaphoreType.DMA,
            pltpu.SemaphoreType.DMA,
            pltpu.SemaphoreType.DMA,
        ],
    )
    def k(patch_hbm, audio_hbm, rowbase_hbm, out_patch, out_audio,
          idx_v, buf0, buf1, abuf, g0, g1, s0, s1, asem):
        wid = lax.axis_index("s") * 2 + lax.axis_index("c")
        pltpu.sync_copy(rowbase_hbm, idx_v)
        lanes = lax.iota(jnp.int32, _L)

        @pl.when(wid < AG)
        def _():
            rows = plsc.load_gather(idx_v, [wid * _L + lanes])
            pltpu.async_copy(audio_hbm.at[rows], abuf, asem).wait()
            pltpu.sync_copy(abuf, out_audio.at[pl.ds(wid * _L, _L)])

        bufs = (buf0, buf1)
        gsems = (g0, g1)
        ssems = (s0, s1)
        base = wid * PPW
        scatters = [None] * NG
        for j in range(NG):
            sel = j % 2
            p = base + j * _L + lanes
            bk = p >> LG
            g = p & (G - 1)
            t2 = plsc.load_gather(idx_v, [bk])
            src = (t2 << LG) + g
            if j >= 2:
                scatters[j - 2].wait()
            pltpu.async_copy(patch_hbm.at[src], bufs[sel], gsems[sel]).wait()
            scatters[j] = pltpu.async_copy(
                bufs[sel], out_patch.at[pl.ds(base + j * _L, _L)], ssems[sel])
        if NG >= 2:
            scatters[NG - 2].wait()
        scatters[NG - 1].wait()

    return k


def kernel(top_k_index_sort, patch_feat, audio_feat):
    B, T, N, C = patch_feat.shape
    K = top_k_index_sort.shape[-1]
    LG = 6
    G = 1 << LG
    d = (N * C) // G
    idx = top_k_index_sort.reshape(B, K).astype(jnp.int32)
    rowbase = (jnp.arange(B, dtype=jnp.int32)[:, None] * T + idx).reshape(B * K)
    patch_view = patch_feat.reshape(B * T * G, d)
    audio_view = audio_feat.reshape(B * T, C)
    out_p, out_a = _build(B, T, N, C, K, LG)(patch_view, audio_view, rowbase)
    return out_p.reshape(B, K, N, C), out_a.reshape(B, K, C)
